# Initial kernel scaffold; baseline (speedup 1.0000x reference)
import jax
import jax.numpy as jnp
from jax.experimental import pallas as pl


def kernel(edge_index, edge_type, weight1, root1, bias1, weight2, root2, bias2):
    raise NotImplementedError("write your pallas kernel here")


# speedup vs baseline: 29.4240x; 29.4240x over previous
import functools

import jax
import jax.numpy as jnp
from jax import lax
from jax.experimental import pallas as pl
from jax.experimental.pallas import tpu as pltpu
from jax.experimental.pallas import tpu_sc as plsc

N = 10000
R = 8
E = 320000
HID = 128
OUT = 128

NC = 2
NS = 16
NW = NC * NS
EPW = E // NW
EPC = E // NS
CH = 48
NCHUNK = 210
NR_PAD = 81920
MSL = NR_PAD // NS

_mesh = plsc.VectorSubcoreMesh(core_axis_name="c", subcore_axis_name="s")
_params = pltpu.CompilerParams(
    needs_layout_passes=False, use_tc_tiling_on_sc=False)


def _zeros16f():
  return jnp.zeros((16,), jnp.float32)


@functools.partial(
    pl.kernel,
    out_type=(
        jax.ShapeDtypeStruct((NW, NCHUNK, CH), jnp.int32),
        jax.ShapeDtypeStruct((NW, NCHUNK, CH), jnp.int32),
        jax.ShapeDtypeStruct((NW, NCHUNK, CH), jnp.float32),
    ),
    mesh=_mesh,
    compiler_params=_params,
    scratch_types=dict(
        ebuf_a=pltpu.VMEM((EPC,), jnp.int32),
        ebuf_b=pltpu.VMEM((EPC,), jnp.int32),
        ebuf_c=pltpu.VMEM((EPW,), jnp.int32),
        srow=pltpu.VMEM((25, 80), jnp.int32),
        ones80=pltpu.VMEM((80,), jnp.float32),
        ibuf=pltpu.VMEM((MSL,), jnp.float32),
        srow2=pltpu.VMEM((NCHUNK, CH), jnp.int32),
        grow=pltpu.VMEM((NCHUNK, CH), jnp.int32),
        drow=pltpu.VMEM((NCHUNK, CH), jnp.int32),
        erow=pltpu.VMEM((NCHUNK, CH), jnp.float32),
        cnt_sp=pltpu.VMEM_SHARED((NR_PAD,), jnp.float32),
        sem=pltpu.SemaphoreType.DMA,
    ),
)
def _prep(src_hbm, dst_hbm, typ_hbm, gidx_hbm, dstp_hbm, esc_hbm,
          ebuf_a, ebuf_b, ebuf_c, srow, ones80, ibuf, srow2, grow, drow,
          erow, cnt_sp, sem):
  cid = lax.axis_index("c")
  sid = lax.axis_index("s")
  wid = cid * NS + sid
  sl = sid * MSL

  def _zi(i, carry):
    ibuf[pl.ds(i * 16, 16)] = _zeros16f()
    return carry
  lax.fori_loop(0, MSL // 16, _zi, 0)
  for j in range(80 // 16):
    ones80[pl.ds(j * 16, 16)] = jnp.ones((16,), jnp.float32)
  pltpu.sync_copy(ibuf, cnt_sp.at[pl.ds(sl, MSL)])
  plsc.subcore_barrier()

  pltpu.sync_copy(dst_hbm.at[pl.ds(sid * EPC, EPC)], ebuf_a)
  pltpu.sync_copy(typ_hbm.at[pl.ds(sid * EPC, EPC)], ebuf_b)

  def _count_chunk(cc, carry):
    def _seg(k, c2):
      d16 = ebuf_a[pl.ds(cc * 2000 + k * 16, 16)]
      t16 = ebuf_b[pl.ds(cc * 2000 + k * 16, 16)]
      srow[k // 5, pl.ds((k % 5) * 16, 16)] = d16 * R + t16
      return c2
    lax.fori_loop(0, 125, _seg, 0)

    descs = [pltpu.async_copy(ones80, cnt_sp.at[srow.at[j]], sem, add=True)
             for j in range(25)]
    for d in descs:
      d.wait()
    return carry
  lax.fori_loop(0, EPC // 2000, _count_chunk, 0)
  plsc.subcore_barrier()

  pltpu.sync_copy(cnt_sp.at[pl.ds(sl, MSL)], ibuf)

  def _inv(i, carry):
    v = ibuf[pl.ds(i * 16, 16)]
    ibuf[pl.ds(i * 16, 16)] = 1.0 / jnp.maximum(v, 1.0)
    return carry
  lax.fori_loop(0, MSL // 16, _inv, 0)
  pltpu.sync_copy(ibuf, cnt_sp.at[pl.ds(sl, MSL)])
  plsc.subcore_barrier()

  pltpu.sync_copy(src_hbm.at[pl.ds(wid * EPW, EPW)], ebuf_c)
  pltpu.sync_copy(dst_hbm.at[pl.ds(wid * EPW, EPW)], ebuf_a.at[pl.ds(0, EPW)])
  pltpu.sync_copy(typ_hbm.at[pl.ds(wid * EPW, EPW)], ebuf_b.at[pl.ds(0, EPW)])

  def _emit(k, carry):
    row = k // 3
    col = (k % 3) * 16
    s16 = ebuf_c[pl.ds(k * 16, 16)]
    d16 = ebuf_a[pl.ds(k * 16, 16)]
    t16 = ebuf_b[pl.ds(k * 16, 16)]
    srow2[row, pl.ds(col, 16)] = d16 * R + t16
    grow[row, pl.ds(col, 16)] = t16 * N + s16
    drow[row, pl.ds(col, 16)] = d16
    return carry
  lax.fori_loop(0, EPW // 16, _emit, 0)

  z16i = jnp.zeros((16,), jnp.int32)
  for buf in (srow2, grow, drow):
    buf[NCHUNK - 2, pl.ds(16, 16)] = z16i
    buf[NCHUNK - 2, pl.ds(32, 16)] = z16i
    for j in range(CH // 16):
      buf[NCHUNK - 1, pl.ds(j * 16, 16)] = z16i

  def _egather(tt, carry):
    descs = [pltpu.async_copy(cnt_sp.at[srow2.at[tt * 7 + i]],
                              erow.at[tt * 7 + i], sem)
             for i in range(7)]
    for d in descs:
      d.wait()
    return carry
  lax.fori_loop(0, NCHUNK // 7, _egather, 0)

  erow[NCHUNK - 2, pl.ds(16, 16)] = _zeros16f()
  erow[NCHUNK - 2, pl.ds(32, 16)] = _zeros16f()
  for j in range(CH // 16):
    erow[NCHUNK - 1, pl.ds(j * 16, 16)] = _zeros16f()

  pltpu.sync_copy(grow, gidx_hbm.at[wid])
  pltpu.sync_copy(drow, dstp_hbm.at[wid])
  pltpu.sync_copy(erow, esc_hbm.at[wid])


@functools.partial(
    pl.kernel,
    out_type=jax.ShapeDtypeStruct((NC, N, HID), jnp.float32),
    mesh=_mesh,
    compiler_params=_params,
    scratch_types=dict(
        gidx_t=pltpu.VMEM((NCHUNK, CH), jnp.int32),
        dst_t=pltpu.VMEM((NCHUNK, CH), jnp.int32),
        esc_t=pltpu.VMEM((NCHUNK, CH), jnp.float32),
        b0=pltpu.VMEM((CH, HID), jnp.float32),
        b1=pltpu.VMEM((CH, HID), jnp.float32),
        b2=pltpu.VMEM((CH, HID), jnp.float32),
        acc_sp=pltpu.VMEM_SHARED((N, HID), jnp.float32),
        sg0=pltpu.SemaphoreType.DMA,
        sg1=pltpu.SemaphoreType.DMA,
        sg2=pltpu.SemaphoreType.DMA,
        ss0=pltpu.SemaphoreType.DMA,
        ss1=pltpu.SemaphoreType.DMA,
        ss2=pltpu.SemaphoreType.DMA,
    ),
)
def _agg(table_hbm, gidx_hbm, dstp_hbm, esc_hbm, out_hbm,
         gidx_t, dst_t, esc_t, b0, b1, b2, acc_sp,
         sg0, sg1, sg2, ss0, ss1, ss2):
  cid = lax.axis_index("c")
  sid = lax.axis_index("s")
  wid = cid * NS + sid
  bufs = (b0, b1, b2)
  gsems = (sg0, sg1, sg2)
  ssems = (ss0, ss1, ss2)
  base = sid * 640

  def _z(i, carry):
    for j in range(HID // 16):
      b0[i, pl.ds(j * 16, 16)] = _zeros16f()
    return carry
  lax.fori_loop(0, CH, _z, 0)

  @pl.when(sid < NS - 1)
  def _zero_full():
    for q in range(640 // CH):
      pltpu.sync_copy(b0, acc_sp.at[pl.ds(base + q * CH, CH)])
    pltpu.sync_copy(b0.at[pl.ds(0, 16)], acc_sp.at[pl.ds(base + 624, 16)])

  @pl.when(sid == NS - 1)
  def _zero_last():
    for q in range(400 // CH):
      pltpu.sync_copy(b0, acc_sp.at[pl.ds(base + q * CH, CH)])
    pltpu.sync_copy(b0.at[pl.ds(0, 16)], acc_sp.at[pl.ds(base + 384, 16)])

  plsc.subcore_barrier()

  pltpu.sync_copy(gidx_hbm.at[wid], gidx_t)
  pltpu.sync_copy(dstp_hbm.at[wid], dst_t)
  pltpu.sync_copy(esc_hbm.at[wid], esc_t)

  def _fire(c, buf, sem):
    pltpu.async_copy(table_hbm.at[gidx_t.at[c]], buf, sem)

  def _scale(c, buf):
    def _body(g, carry):
      ev = esc_t[c, pl.ds(g * 16, 16)]
      for l in range(16):
        s = ev[l]
        r0 = g * 16 + l
        for j in range(HID // 16):
          buf[r0, pl.ds(j * 16, 16)] = buf[r0, pl.ds(j * 16, 16)] * s
      return carry
    lax.fori_loop(0, CH // 16, _body, 0)

  _fire(0, b0, sg0)
  _fire(1, b1, sg1)
  _fire(2, b2, sg2)

  def _triple(tt, carry):
    for i in range(3):
      c = tt * 3 + i
      buf, gsem, ssem = bufs[i], gsems[i], ssems[i]
      pbuf, pgsem, pssem = bufs[i - 1], gsems[i - 1], ssems[i - 1]

      pltpu.make_async_copy(table_hbm.at[gidx_t.at[c]], buf, gsem).wait()
      _scale(c, buf)

      @pl.when(c >= 1)
      def _recycle_prev():
        pltpu.make_async_copy(pbuf, acc_sp.at[dst_t.at[c - 1]], pssem).wait()

        @pl.when(c + 2 < NCHUNK)
        def _refill():
          pltpu.async_copy(table_hbm.at[gidx_t.at[c + 2]], pbuf, pgsem)

      pltpu.async_copy(buf, acc_sp.at[dst_t.at[c]], ssem, add=True)
    return carry
  lax.fori_loop(0, NCHUNK // 3, _triple, 0)

  pltpu.make_async_copy(b2, acc_sp.at[dst_t.at[NCHUNK - 1]], ss2).wait()

  plsc.subcore_barrier()

  @pl.when(sid < NS - 1)
  def _drain_full():
    pltpu.sync_copy(acc_sp.at[pl.ds(base, 640)],
                    out_hbm.at[cid, pl.ds(base, 640)])

  @pl.when(sid == NS - 1)
  def _drain_last():
    pltpu.sync_copy(acc_sp.at[pl.ds(base, 400)],
                    out_hbm.at[cid, pl.ds(base, 400)])


NB = 10
BN = N // NB


def _tc1_body(a_ref, r1_ref, b1_ref, w2_ref, rt2_ref, xw_ref, xr_ref):
  x = jnp.maximum(a_ref[0] + a_ref[1] + r1_ref[...] + b1_ref[...], 0.0)
  xr_ref[...] = jnp.dot(x, rt2_ref[...], preferred_element_type=jnp.float32)
  for r in range(R):
    xw_ref[r] = jnp.dot(x, w2_ref[r], preferred_element_type=jnp.float32)


_tc1 = pl.pallas_call(
    _tc1_body,
    grid=(NB,),
    in_specs=[
        pl.BlockSpec((NC, BN, HID), lambda i: (0, i, 0)),
        pl.BlockSpec((BN, HID), lambda i: (i, 0)),
        pl.BlockSpec((1, HID), lambda i: (0, 0)),
        pl.BlockSpec((R, HID, OUT), lambda i: (0, 0, 0)),
        pl.BlockSpec((HID, OUT), lambda i: (0, 0)),
    ],
    out_specs=[
        pl.BlockSpec((R, BN, OUT), lambda i: (0, i, 0)),
        pl.BlockSpec((BN, OUT), lambda i: (i, 0)),
    ],
    out_shape=[
        jax.ShapeDtypeStruct((R, N, OUT), jnp.float32),
        jax.ShapeDtypeStruct((N, OUT), jnp.float32),
    ],
)


def _tc2_body(a_ref, xr_ref, b2_ref, o_ref):
  o_ref[...] = a_ref[0] + a_ref[1] + xr_ref[...] + b2_ref[...]


_tc2 = pl.pallas_call(
    _tc2_body,
    grid=(NB,),
    in_specs=[
        pl.BlockSpec((NC, BN, OUT), lambda i: (0, i, 0)),
        pl.BlockSpec((BN, OUT), lambda i: (i, 0)),
        pl.BlockSpec((1, OUT), lambda i: (0, 0)),
    ],
    out_specs=pl.BlockSpec((BN, OUT), lambda i: (i, 0)),
    out_shape=jax.ShapeDtypeStruct((N, OUT), jnp.float32),
)


def kernel(edge_index, edge_type, weight1, root1, bias1, weight2, root2, bias2):
  src = edge_index[0]
  dst = edge_index[1]
  gidx, dstp, esc = _prep(src, dst, edge_type)
  acc1 = _agg(weight1.reshape(R * N, HID), gidx, dstp, esc)
  xw, xr = _tc1(acc1, root1, bias1.reshape(1, HID), weight2, root2)
  acc2 = _agg(xw.reshape(R * N, OUT), gidx, dstp, esc)
  return _tc2(acc2, xr, bias2.reshape(1, OUT))


# speedup vs baseline: 39.9616x; 1.3581x over previous
import functools

import jax
import jax.numpy as jnp
from jax import lax
from jax.experimental import pallas as pl
from jax.experimental.pallas import tpu as pltpu
from jax.experimental.pallas import tpu_sc as plsc

N = 10000
R = 8
E = 320000
HID = 128
OUT = 128

NC = 2
NS = 16
NW = NC * NS
EPW = E // NW
EPC = E // NS
CH = 80
NCHUNK = EPW // CH
NR_PAD = 81920
MSL = NR_PAD // NS

_mesh = plsc.VectorSubcoreMesh(core_axis_name="c", subcore_axis_name="s")
_params = pltpu.CompilerParams(
    needs_layout_passes=False, use_tc_tiling_on_sc=False)


def _zeros16f():
  return jnp.zeros((16,), jnp.float32)


@functools.partial(
    pl.kernel,
    out_type=(
        jax.ShapeDtypeStruct((NW, NCHUNK, CH), jnp.int32),
        jax.ShapeDtypeStruct((NW, NCHUNK, CH), jnp.int32),
        jax.ShapeDtypeStruct((NW, NCHUNK, CH), jnp.float32),
    ),
    mesh=_mesh,
    compiler_params=_params,
    scratch_types=dict(
        ebuf_a=pltpu.VMEM((2000,), jnp.int32),
        ebuf_b=pltpu.VMEM((2000,), jnp.int32),
        ebuf_c=pltpu.VMEM((2000,), jnp.int32),
        srow=pltpu.VMEM((25, CH), jnp.int32),
        ones80=pltpu.VMEM((CH,), jnp.float32),
        ibuf=pltpu.VMEM((MSL,), jnp.float32),
        grow=pltpu.VMEM((NCHUNK, CH), jnp.int32),
        drow=pltpu.VMEM((NCHUNK, CH), jnp.int32),
        erow=pltpu.VMEM((NCHUNK, CH), jnp.float32),
        cnt_sp=pltpu.VMEM_SHARED((NR_PAD,), jnp.float32),
        sem=pltpu.SemaphoreType.DMA,
    ),
)
def _prep(src_hbm, dst_hbm, typ_hbm, gidx_hbm, dstp_hbm, esc_hbm,
          ebuf_a, ebuf_b, ebuf_c, srow, ones80, ibuf, grow, drow, erow,
          cnt_sp, sem):
  cid = lax.axis_index("c")
  sid = lax.axis_index("s")
  wid = cid * NS + sid
  sl = sid * MSL

  def _zi(i, carry):
    ibuf[pl.ds(i * 16, 16)] = _zeros16f()
    return carry
  lax.fori_loop(0, MSL // 16, _zi, 0)
  for j in range(CH // 16):
    ones80[pl.ds(j * 16, 16)] = jnp.ones((16,), jnp.float32)
  pltpu.sync_copy(ibuf, cnt_sp.at[pl.ds(sl, MSL)])
  plsc.subcore_barrier()

  def _count_chunk(cc, carry):
    base = sid * EPC + cc * 2000
    pltpu.sync_copy(dst_hbm.at[pl.ds(base, 2000)], ebuf_a)
    pltpu.sync_copy(typ_hbm.at[pl.ds(base, 2000)], ebuf_b)

    def _seg(k, c2):
      d16 = ebuf_a[pl.ds(k * 16, 16)]
      t16 = ebuf_b[pl.ds(k * 16, 16)]
      srow[k // 5, pl.ds((k % 5) * 16, 16)] = d16 * R + t16
      return c2
    lax.fori_loop(0, 125, _seg, 0)

    descs = [pltpu.async_copy(ones80, cnt_sp.at[srow.at[j]], sem, add=True)
             for j in range(25)]
    for d in descs:
      d.wait()
    return carry
  lax.fori_loop(0, EPC // 2000, _count_chunk, 0)
  plsc.subcore_barrier()

  pltpu.sync_copy(cnt_sp.at[pl.ds(sl, MSL)], ibuf)

  def _inv(i, carry):
    v = ibuf[pl.ds(i * 16, 16)]
    ibuf[pl.ds(i * 16, 16)] = 1.0 / jnp.maximum(v, 1.0)
    return carry
  lax.fori_loop(0, MSL // 16, _inv, 0)
  pltpu.sync_copy(ibuf, cnt_sp.at[pl.ds(sl, MSL)])
  plsc.subcore_barrier()

  def _out_chunk(oc, carry):
    base = wid * EPW + oc * 2000
    pltpu.sync_copy(src_hbm.at[pl.ds(base, 2000)], ebuf_a)
    pltpu.sync_copy(dst_hbm.at[pl.ds(base, 2000)], ebuf_c)
    pltpu.sync_copy(typ_hbm.at[pl.ds(base, 2000)], ebuf_b)

    def _emit(k, c2):
      row = oc * 25 + k // 5
      col = (k % 5) * 16
      s16 = ebuf_a[pl.ds(k * 16, 16)]
      d16 = ebuf_c[pl.ds(k * 16, 16)]
      t16 = ebuf_b[pl.ds(k * 16, 16)]
      srow[k // 5, pl.ds(col, 16)] = d16 * R + t16
      grow[row, pl.ds(col, 16)] = t16 * N + s16
      drow[row, pl.ds(col, 16)] = d16
      return c2
    lax.fori_loop(0, 125, _emit, 0)

    descs = [pltpu.async_copy(cnt_sp.at[srow.at[j]], erow.at[oc * 25 + j], sem)
             for j in range(25)]
    for d in descs:
      d.wait()
    return carry
  lax.fori_loop(0, EPW // 2000, _out_chunk, 0)

  pltpu.sync_copy(grow, gidx_hbm.at[wid])
  pltpu.sync_copy(drow, dstp_hbm.at[wid])
  pltpu.sync_copy(erow, esc_hbm.at[wid])


@functools.partial(
    pl.kernel,
    out_type=jax.ShapeDtypeStruct((NC, N, HID), jnp.float32),
    mesh=_mesh,
    compiler_params=_params,
    scratch_types=dict(
        gidx_t=pltpu.VMEM((NCHUNK, CH), jnp.int32),
        dst_t=pltpu.VMEM((NCHUNK, CH), jnp.int32),
        esc_t=pltpu.VMEM((NCHUNK, CH), jnp.float32),
        rows0=pltpu.VMEM((CH, HID), jnp.float32),
        rows1=pltpu.VMEM((CH, HID), jnp.float32),
        acc_sp=pltpu.VMEM_SHARED((N, HID), jnp.float32),
        sg0=pltpu.SemaphoreType.DMA,
        sg1=pltpu.SemaphoreType.DMA,
    ),
)
def _agg(table_hbm, gidx_hbm, dstp_hbm, esc_hbm, out_hbm,
         gidx_t, dst_t, esc_t, rows0, rows1, acc_sp, sg0, sg1):
  cid = lax.axis_index("c")
  sid = lax.axis_index("s")
  wid = cid * NS + sid
  base = sid * 640

  def _z(i, carry):
    for j in range(HID // 16):
      rows0[i, pl.ds(j * 16, 16)] = _zeros16f()
    return carry
  lax.fori_loop(0, CH, _z, 0)

  @pl.when(sid < NS - 1)
  def _zero_full():
    for q in range(640 // CH):
      pltpu.sync_copy(rows0, acc_sp.at[pl.ds(base + q * CH, CH)])

  @pl.when(sid == NS - 1)
  def _zero_last():
    for q in range(400 // CH):
      pltpu.sync_copy(rows0, acc_sp.at[pl.ds(base + q * CH, CH)])

  plsc.subcore_barrier()

  pltpu.sync_copy(gidx_hbm.at[wid], gidx_t)
  pltpu.sync_copy(dstp_hbm.at[wid], dst_t)
  pltpu.sync_copy(esc_hbm.at[wid], esc_t)

  def _fire(c, rows, sem):
    pltpu.async_copy(table_hbm.at[gidx_t.at[c]], rows, sem)

  def _step(c, rows, sem):
    pltpu.make_async_copy(table_hbm.at[gidx_t.at[c]], rows, sem).wait()

    def _scale(g, carry):
      ev = esc_t[c, pl.ds(g * 16, 16)]
      for l in range(16):
        s = ev[l]
        r0 = g * 16 + l
        for j in range(HID // 16):
          rows[r0, pl.ds(j * 16, 16)] = rows[r0, pl.ds(j * 16, 16)] * s
      return carry
    lax.fori_loop(0, CH // 16, _scale, 0, unroll=5)

    pltpu.sync_copy(rows, acc_sp.at[dst_t.at[c]], add=True)

  _fire(0, rows0, sg0)
  _fire(1, rows1, sg1)

  def _pair(cc, carry):
    c0 = cc * 2
    _step(c0, rows0, sg0)
    _fire(c0 + 2, rows0, sg0)
    c1 = c0 + 1

    _step(c1, rows1, sg1)

    @pl.when(c1 + 2 < NCHUNK)
    def _fire_odd():
      _fire(c1 + 2, rows1, sg1)
    return carry
  lax.fori_loop(0, NCHUNK // 2, _pair, 0)
  _step(NCHUNK - 1, rows0, sg0)

  plsc.subcore_barrier()

  @pl.when(sid < NS - 1)
  def _drain_full():
    pltpu.sync_copy(acc_sp.at[pl.ds(base, 640)],
                    out_hbm.at[cid, pl.ds(base, 640)])

  @pl.when(sid == NS - 1)
  def _drain_last():
    pltpu.sync_copy(acc_sp.at[pl.ds(base, 400)],
                    out_hbm.at[cid, pl.ds(base, 400)])


NB = 10
BN = N // NB


def _tc1_body(a_ref, r1_ref, b1_ref, w2_ref, rt2_ref, xw_ref, xr_ref):
  x = jnp.maximum(a_ref[0] + a_ref[1] + r1_ref[...] + b1_ref[...], 0.0)
  xr_ref[...] = jnp.dot(x, rt2_ref[...], preferred_element_type=jnp.float32)
  for r in range(R):
    xw_ref[r] = jnp.dot(x, w2_ref[r], preferred_element_type=jnp.float32)


_tc1 = pl.pallas_call(
    _tc1_body,
    grid=(NB,),
    in_specs=[
        pl.BlockSpec((NC, BN, HID), lambda i: (0, i, 0)),
        pl.BlockSpec((BN, HID), lambda i: (i, 0)),
        pl.BlockSpec((1, HID), lambda i: (0, 0)),
        pl.BlockSpec((R, HID, OUT), lambda i: (0, 0, 0)),
        pl.BlockSpec((HID, OUT), lambda i: (0, 0)),
    ],
    out_specs=[
        pl.BlockSpec((R, BN, OUT), lambda i: (0, i, 0)),
        pl.BlockSpec((BN, OUT), lambda i: (i, 0)),
    ],
    out_shape=[
        jax.ShapeDtypeStruct((R, N, OUT), jnp.float32),
        jax.ShapeDtypeStruct((N, OUT), jnp.float32),
    ],
)


def _tc2_body(a_ref, xr_ref, b2_ref, o_ref):
  o_ref[...] = a_ref[0] + a_ref[1] + xr_ref[...] + b2_ref[...]


_tc2 = pl.pallas_call(
    _tc2_body,
    grid=(NB,),
    in_specs=[
        pl.BlockSpec((NC, BN, OUT), lambda i: (0, i, 0)),
        pl.BlockSpec((BN, OUT), lambda i: (i, 0)),
        pl.BlockSpec((1, OUT), lambda i: (0, 0)),
    ],
    out_specs=pl.BlockSpec((BN, OUT), lambda i: (i, 0)),
    out_shape=jax.ShapeDtypeStruct((N, OUT), jnp.float32),
)


def kernel(edge_index, edge_type, weight1, root1, bias1, weight2, root2, bias2):
  src = edge_index[0]
  dst = edge_index[1]
  gidx, dstp, esc = _prep(src, dst, edge_type)
  acc1 = _agg(weight1.reshape(R * N, HID), gidx, dstp, esc)
  xw, xr = _tc1(acc1, root1, bias1.reshape(1, HID), weight2, root2)
  acc2 = _agg(xw.reshape(R * N, OUT), gidx, dstp, esc)
  return _tc2(acc2, xr, bias2.reshape(1, OUT))


# speedup vs baseline: 42.8116x; 1.0713x over previous
import functools

import jax
import jax.numpy as jnp
from jax import lax
from jax.experimental import pallas as pl
from jax.experimental.pallas import tpu as pltpu
from jax.experimental.pallas import tpu_sc as plsc

N = 10000
R = 8
E = 320000
HID = 128
OUT = 128

NC = 2
NS = 16
NW = NC * NS
EPW = E // NW
EPC = E // NS
CH = 80
NCHUNK = EPW // CH
NR_PAD = 81920
MSL = NR_PAD // NS

_mesh = plsc.VectorSubcoreMesh(core_axis_name="c", subcore_axis_name="s")
_params = pltpu.CompilerParams(
    needs_layout_passes=False, use_tc_tiling_on_sc=False)


def _zeros16f():
  return jnp.zeros((16,), jnp.float32)


@functools.partial(
    pl.kernel,
    out_type=(
        jax.ShapeDtypeStruct((NW, NCHUNK, CH), jnp.int32),
        jax.ShapeDtypeStruct((NW, NCHUNK, CH), jnp.int32),
        jax.ShapeDtypeStruct((NW, NCHUNK, CH), jnp.float32),
    ),
    mesh=_mesh,
    compiler_params=_params,
    scratch_types=dict(
        ebuf_a=pltpu.VMEM((EPC,), jnp.int32),
        ebuf_b=pltpu.VMEM((EPC,), jnp.int32),
        ebuf_c=pltpu.VMEM((EPW,), jnp.int32),
        srow0=pltpu.VMEM((25, CH), jnp.int32),
        srow1=pltpu.VMEM((25, CH), jnp.int32),
        ones80=pltpu.VMEM((CH,), jnp.float32),
        ibuf=pltpu.VMEM((MSL,), jnp.float32),
        srow2=pltpu.VMEM((NCHUNK, CH), jnp.int32),
        grow=pltpu.VMEM((NCHUNK, CH), jnp.int32),
        drow=pltpu.VMEM((NCHUNK, CH), jnp.int32),
        erow=pltpu.VMEM((NCHUNK, CH), jnp.float32),
        cnt_sp=pltpu.VMEM_SHARED((NR_PAD,), jnp.float32),
        sem=pltpu.SemaphoreType.DMA,
        sem1=pltpu.SemaphoreType.DMA,
    ),
)
def _prep(src_hbm, dst_hbm, typ_hbm, gidx_hbm, dstp_hbm, esc_hbm,
          ebuf_a, ebuf_b, ebuf_c, srow0, srow1, ones80, ibuf, srow2,
          grow, drow, erow, cnt_sp, sem, sem1):
  cid = lax.axis_index("c")
  sid = lax.axis_index("s")
  wid = cid * NS + sid
  sl = sid * MSL

  def _zi(i, carry):
    ibuf[pl.ds(i * 16, 16)] = _zeros16f()
    return carry
  lax.fori_loop(0, MSL // 16, _zi, 0)
  for j in range(CH // 16):
    ones80[pl.ds(j * 16, 16)] = jnp.ones((16,), jnp.float32)
  pltpu.sync_copy(ibuf, cnt_sp.at[pl.ds(sl, MSL)])
  plsc.subcore_barrier()

  pltpu.sync_copy(dst_hbm.at[pl.ds(sid * EPC, EPC)], ebuf_a)
  pltpu.sync_copy(typ_hbm.at[pl.ds(sid * EPC, EPC)], ebuf_b)

  def _build_segs(cc, srow):
    def _seg(k, c2):
      d16 = ebuf_a[pl.ds(cc * 2000 + k * 16, 16)]
      t16 = ebuf_b[pl.ds(cc * 2000 + k * 16, 16)]
      srow[k // 5, pl.ds((k % 5) * 16, 16)] = d16 * R + t16
      return c2
    lax.fori_loop(0, 125, _seg, 0, unroll=5)

  def _fire_counts(srow, csem):
    return [pltpu.async_copy(ones80, cnt_sp.at[srow.at[j]], csem, add=True)
            for j in range(25)]

  def _drain_counts(srow, csem):
    for j in range(25):
      pltpu.make_async_copy(ones80, cnt_sp.at[srow.at[j]], csem).wait()

  def _count_pair(cc2, carry):
    cc = cc2 * 2
    _build_segs(cc, srow0)

    @pl.when(cc2 > 0)
    def _drain_prev_odd():
      _drain_counts(srow1, sem1)
    _fire_counts(srow0, sem)

    _build_segs(cc + 1, srow1)
    _drain_counts(srow0, sem)
    _fire_counts(srow1, sem1)
    return carry
  lax.fori_loop(0, EPC // 4000, _count_pair, 0)
  _drain_counts(srow1, sem1)
  plsc.subcore_barrier()

  pltpu.sync_copy(cnt_sp.at[pl.ds(sl, MSL)], ibuf)

  def _inv(i, carry):
    v = ibuf[pl.ds(i * 16, 16)]
    ibuf[pl.ds(i * 16, 16)] = 1.0 / jnp.maximum(v, 1.0)
    return carry
  lax.fori_loop(0, MSL // 16, _inv, 0)
  pltpu.sync_copy(ibuf, cnt_sp.at[pl.ds(sl, MSL)])
  plsc.subcore_barrier()

  pltpu.sync_copy(src_hbm.at[pl.ds(wid * EPW, EPW)], ebuf_c)
  pltpu.sync_copy(dst_hbm.at[pl.ds(wid * EPW, EPW)], ebuf_a.at[pl.ds(0, EPW)])
  pltpu.sync_copy(typ_hbm.at[pl.ds(wid * EPW, EPW)], ebuf_b.at[pl.ds(0, EPW)])

  def _emit(k, carry):
    row = k // 5
    col = (k % 5) * 16
    s16 = ebuf_c[pl.ds(k * 16, 16)]
    d16 = ebuf_a[pl.ds(k * 16, 16)]
    t16 = ebuf_b[pl.ds(k * 16, 16)]
    srow2[row, pl.ds(col, 16)] = d16 * R + t16
    grow[row, pl.ds(col, 16)] = t16 * N + s16
    drow[row, pl.ds(col, 16)] = d16
    return carry
  lax.fori_loop(0, EPW // 16, _emit, 0, unroll=5)

  def _egather(tt, carry):
    descs = [pltpu.async_copy(cnt_sp.at[srow2.at[tt * 5 + i]],
                              erow.at[tt * 5 + i], sem)
             for i in range(5)]
    for d in descs:
      d.wait()
    return carry
  lax.fori_loop(0, NCHUNK // 5, _egather, 0)

  pltpu.sync_copy(grow, gidx_hbm.at[wid])
  pltpu.sync_copy(drow, dstp_hbm.at[wid])
  pltpu.sync_copy(erow, esc_hbm.at[wid])


@functools.partial(
    pl.kernel,
    out_type=jax.ShapeDtypeStruct((NC, N, HID), jnp.float32),
    mesh=_mesh,
    compiler_params=_params,
    scratch_types=dict(
        gidx_t=pltpu.VMEM((NCHUNK, CH), jnp.int32),
        dst_t=pltpu.VMEM((NCHUNK, CH), jnp.int32),
        esc_t=pltpu.VMEM((NCHUNK, CH), jnp.float32),
        rows0=pltpu.VMEM((CH, HID), jnp.float32),
        rows1=pltpu.VMEM((CH, HID), jnp.float32),
        acc_sp=pltpu.VMEM_SHARED((N, HID), jnp.float32),
        sg0=pltpu.SemaphoreType.DMA,
        sg1=pltpu.SemaphoreType.DMA,
    ),
)
def _agg(table_hbm, gidx_hbm, dstp_hbm, esc_hbm, out_hbm,
         gidx_t, dst_t, esc_t, rows0, rows1, acc_sp, sg0, sg1):
  cid = lax.axis_index("c")
  sid = lax.axis_index("s")
  wid = cid * NS + sid
  base = sid * 640

  def _z(i, carry):
    for j in range(HID // 16):
      rows0[i, pl.ds(j * 16, 16)] = _zeros16f()
    return carry
  lax.fori_loop(0, CH, _z, 0)

  @pl.when(sid < NS - 1)
  def _zero_full():
    for q in range(640 // CH):
      pltpu.sync_copy(rows0, acc_sp.at[pl.ds(base + q * CH, CH)])

  @pl.when(sid == NS - 1)
  def _zero_last():
    for q in range(400 // CH):
      pltpu.sync_copy(rows0, acc_sp.at[pl.ds(base + q * CH, CH)])

  plsc.subcore_barrier()

  pltpu.sync_copy(gidx_hbm.at[wid], gidx_t)
  pltpu.sync_copy(dstp_hbm.at[wid], dst_t)
  pltpu.sync_copy(esc_hbm.at[wid], esc_t)

  def _fire(c, rows, sem):
    pltpu.async_copy(table_hbm.at[gidx_t.at[c]], rows, sem)

  def _step(c, rows, sem):
    pltpu.make_async_copy(table_hbm.at[gidx_t.at[c]], rows, sem).wait()

    def _scale(g, carry):
      ev = esc_t[c, pl.ds(g * 16, 16)]
      for l in range(16):
        s = ev[l]
        r0 = g * 16 + l
        for j in range(HID // 16):
          rows[r0, pl.ds(j * 16, 16)] = rows[r0, pl.ds(j * 16, 16)] * s
      return carry
    lax.fori_loop(0, CH // 16, _scale, 0)

    pltpu.sync_copy(rows, acc_sp.at[dst_t.at[c]], add=True)

  _fire(0, rows0, sg0)
  _fire(1, rows1, sg1)

  def _pair(cc, carry):
    c0 = cc * 2
    _step(c0, rows0, sg0)
    _fire(c0 + 2, rows0, sg0)
    c1 = c0 + 1

    _step(c1, rows1, sg1)

    @pl.when(c1 + 2 < NCHUNK)
    def _fire_odd():
      _fire(c1 + 2, rows1, sg1)
    return carry
  lax.fori_loop(0, NCHUNK // 2, _pair, 0)
  _step(NCHUNK - 1, rows0, sg0)

  plsc.subcore_barrier()

  @pl.when(sid < NS - 1)
  def _drain_full():
    pltpu.sync_copy(acc_sp.at[pl.ds(base, 640)],
                    out_hbm.at[cid, pl.ds(base, 640)])

  @pl.when(sid == NS - 1)
  def _drain_last():
    pltpu.sync_copy(acc_sp.at[pl.ds(base, 400)],
                    out_hbm.at[cid, pl.ds(base, 400)])


NB = 10
BN = N // NB


def _tc1_body(a_ref, r1_ref, b1_ref, w2_ref, rt2_ref, xw_ref, xr_ref):
  x = jnp.maximum(a_ref[0] + a_ref[1] + r1_ref[...] + b1_ref[...], 0.0)
  xr_ref[...] = jnp.dot(x, rt2_ref[...], preferred_element_type=jnp.float32)
  for r in range(R):
    xw_ref[r] = jnp.dot(x, w2_ref[r], preferred_element_type=jnp.float32)


_tc1 = pl.pallas_call(
    _tc1_body,
    grid=(NB,),
    in_specs=[
        pl.BlockSpec((NC, BN, HID), lambda i: (0, i, 0)),
        pl.BlockSpec((BN, HID), lambda i: (i, 0)),
        pl.BlockSpec((1, HID), lambda i: (0, 0)),
        pl.BlockSpec((R, HID, OUT), lambda i: (0, 0, 0)),
        pl.BlockSpec((HID, OUT), lambda i: (0, 0)),
    ],
    out_specs=[
        pl.BlockSpec((R, BN, OUT), lambda i: (0, i, 0)),
        pl.BlockSpec((BN, OUT), lambda i: (i, 0)),
    ],
    out_shape=[
        jax.ShapeDtypeStruct((R, N, OUT), jnp.float32),
        jax.ShapeDtypeStruct((N, OUT), jnp.float32),
    ],
)


def _tc2_body(a_ref, xr_ref, b2_ref, o_ref):
  o_ref[...] = a_ref[0] + a_ref[1] + xr_ref[...] + b2_ref[...]


_tc2 = pl.pallas_call(
    _tc2_body,
    grid=(NB,),
    in_specs=[
        pl.BlockSpec((NC, BN, OUT), lambda i: (0, i, 0)),
        pl.BlockSpec((BN, OUT), lambda i: (i, 0)),
        pl.BlockSpec((1, OUT), lambda i: (0, 0)),
    ],
    out_specs=pl.BlockSpec((BN, OUT), lambda i: (i, 0)),
    out_shape=jax.ShapeDtypeStruct((N, OUT), jnp.float32),
)


def kernel(edge_index, edge_type, weight1, root1, bias1, weight2, root2, bias2):
  src = edge_index[0]
  dst = edge_index[1]
  gidx, dstp, esc = _prep(src, dst, edge_type)
  acc1 = _agg(weight1.reshape(R * N, HID), gidx, dstp, esc)
  xw, xr = _tc1(acc1, root1, bias1.reshape(1, HID), weight2, root2)
  acc2 = _agg(xw.reshape(R * N, OUT), gidx, dstp, esc)
  return _tc2(acc2, xr, bias2.reshape(1, OUT))


# speedup vs baseline: 43.5438x; 1.0171x over previous
import functools

import jax
import jax.numpy as jnp
from jax import lax
from jax.experimental import pallas as pl
from jax.experimental.pallas import tpu as pltpu
from jax.experimental.pallas import tpu_sc as plsc

N = 10000
R = 8
E = 320000
HID = 128
OUT = 128

NC = 2
NS = 16
NW = NC * NS
EPW = E // NW
EPC = E // NS
CH = 80
NCHUNK = EPW // CH
NR_PAD = 81920
MSL = NR_PAD // NS

_mesh = plsc.VectorSubcoreMesh(core_axis_name="c", subcore_axis_name="s")
_params = pltpu.CompilerParams(
    needs_layout_passes=False, use_tc_tiling_on_sc=False)


def _zeros16f():
  return jnp.zeros((16,), jnp.float32)


@functools.partial(
    pl.kernel,
    out_type=(
        jax.ShapeDtypeStruct((NW, NCHUNK, CH), jnp.int32),
        jax.ShapeDtypeStruct((NW, NCHUNK, CH), jnp.int32),
        jax.ShapeDtypeStruct((NW, NCHUNK, CH), jnp.float32),
    ),
    mesh=_mesh,
    compiler_params=_params,
    scratch_types=dict(
        ebuf_a=pltpu.VMEM((EPC,), jnp.int32),
        ebuf_b=pltpu.VMEM((EPC,), jnp.int32),
        ebuf_c=pltpu.VMEM((EPW,), jnp.int32),
        srow0=pltpu.VMEM((25, CH), jnp.int32),
        srow1=pltpu.VMEM((25, CH), jnp.int32),
        ones80=pltpu.VMEM((CH,), jnp.float32),
        ibuf=pltpu.VMEM((MSL,), jnp.float32),
        srow2=pltpu.VMEM((NCHUNK, CH), jnp.int32),
        grow=pltpu.VMEM((NCHUNK, CH), jnp.int32),
        drow=pltpu.VMEM((NCHUNK, CH), jnp.int32),
        erow=pltpu.VMEM((NCHUNK, CH), jnp.float32),
        cnt_sp=pltpu.VMEM_SHARED((NR_PAD,), jnp.float32),
        sem=pltpu.SemaphoreType.DMA,
        sem1=pltpu.SemaphoreType.DMA,
    ),
)
def _prep(src_hbm, dst_hbm, typ_hbm, gidx_hbm, dstp_hbm, esc_hbm,
          ebuf_a, ebuf_b, ebuf_c, srow0, srow1, ones80, ibuf, srow2,
          grow, drow, erow, cnt_sp, sem, sem1):
  cid = lax.axis_index("c")
  sid = lax.axis_index("s")
  wid = cid * NS + sid
  sl = sid * MSL

  def _zi(i, carry):
    ibuf[pl.ds(i * 16, 16)] = _zeros16f()
    return carry
  lax.fori_loop(0, MSL // 16, _zi, 0)
  for j in range(CH // 16):
    ones80[pl.ds(j * 16, 16)] = jnp.ones((16,), jnp.float32)
  pltpu.sync_copy(ibuf, cnt_sp.at[pl.ds(sl, MSL)])
  plsc.subcore_barrier()

  pltpu.sync_copy(dst_hbm.at[pl.ds(sid * EPC, EPC)], ebuf_a)
  pltpu.sync_copy(typ_hbm.at[pl.ds(sid * EPC, EPC)], ebuf_b)

  def _build_segs(cc, srow):
    def _seg(k, c2):
      d16 = ebuf_a[pl.ds(cc * 2000 + k * 16, 16)]
      t16 = ebuf_b[pl.ds(cc * 2000 + k * 16, 16)]
      srow[k // 5, pl.ds((k % 5) * 16, 16)] = d16 * R + t16
      return c2
    lax.fori_loop(0, 125, _seg, 0, unroll=5)

  def _fire_counts(srow, csem):
    return [pltpu.async_copy(ones80, cnt_sp.at[srow.at[j]], csem, add=True)
            for j in range(25)]

  def _drain_counts(srow, csem):
    for j in range(25):
      pltpu.make_async_copy(ones80, cnt_sp.at[srow.at[j]], csem).wait()

  def _count_pair(cc2, carry):
    cc = cc2 * 2
    _build_segs(cc, srow0)

    @pl.when(cc2 > 0)
    def _drain_prev_odd():
      _drain_counts(srow1, sem1)
    _fire_counts(srow0, sem)

    _build_segs(cc + 1, srow1)
    _drain_counts(srow0, sem)
    _fire_counts(srow1, sem1)
    return carry
  lax.fori_loop(0, EPC // 4000, _count_pair, 0)
  _drain_counts(srow1, sem1)
  plsc.subcore_barrier()

  pltpu.sync_copy(cnt_sp.at[pl.ds(sl, MSL)], ibuf)

  def _inv(i, carry):
    v = ibuf[pl.ds(i * 16, 16)]
    ibuf[pl.ds(i * 16, 16)] = 1.0 / jnp.maximum(v, 1.0)
    return carry
  lax.fori_loop(0, MSL // 16, _inv, 0)
  pltpu.sync_copy(ibuf, cnt_sp.at[pl.ds(sl, MSL)])
  plsc.subcore_barrier()

  pltpu.sync_copy(src_hbm.at[pl.ds(wid * EPW, EPW)], ebuf_c)
  pltpu.sync_copy(dst_hbm.at[pl.ds(wid * EPW, EPW)], ebuf_a.at[pl.ds(0, EPW)])
  pltpu.sync_copy(typ_hbm.at[pl.ds(wid * EPW, EPW)], ebuf_b.at[pl.ds(0, EPW)])

  def _emit(k, carry):
    row = k // 5
    col = (k % 5) * 16
    s16 = ebuf_c[pl.ds(k * 16, 16)]
    d16 = ebuf_a[pl.ds(k * 16, 16)]
    t16 = ebuf_b[pl.ds(k * 16, 16)]
    srow2[row, pl.ds(col, 16)] = d16 * R + t16
    grow[row, pl.ds(col, 16)] = t16 * N + s16
    drow[row, pl.ds(col, 16)] = d16
    return carry
  lax.fori_loop(0, EPW // 16, _emit, 0, unroll=5)

  def _egather(tt, carry):
    descs = [pltpu.async_copy(cnt_sp.at[srow2.at[tt * 5 + i]],
                              erow.at[tt * 5 + i], sem)
             for i in range(5)]
    for d in descs:
      d.wait()
    return carry
  lax.fori_loop(0, NCHUNK // 5, _egather, 0)

  pltpu.sync_copy(grow, gidx_hbm.at[wid])
  pltpu.sync_copy(drow, dstp_hbm.at[wid])
  pltpu.sync_copy(erow, esc_hbm.at[wid])


@functools.partial(
    pl.kernel,
    out_type=jax.ShapeDtypeStruct((NC, N, HID), jnp.float32),
    mesh=_mesh,
    compiler_params=_params,
    scratch_types=dict(
        gidx_t=pltpu.VMEM((NCHUNK, CH), jnp.int32),
        dst_t=pltpu.VMEM((NCHUNK, CH), jnp.int32),
        esc_t=pltpu.VMEM((NCHUNK, CH), jnp.float32),
        rows0=pltpu.VMEM((CH, HID), jnp.float32),
        rows1=pltpu.VMEM((CH, HID), jnp.float32),
        acc_sp=pltpu.VMEM_SHARED((N, HID), jnp.float32),
        sg0=pltpu.SemaphoreType.DMA,
        sg1=pltpu.SemaphoreType.DMA,
    ),
)
def _agg(table_hbm, gidx_hbm, dstp_hbm, esc_hbm, out_hbm,
         gidx_t, dst_t, esc_t, rows0, rows1, acc_sp, sg0, sg1):
  cid = lax.axis_index("c")
  sid = lax.axis_index("s")
  wid = cid * NS + sid
  base = sid * 640

  pltpu.async_copy(gidx_hbm.at[wid], gidx_t, sg1)
  pltpu.async_copy(dstp_hbm.at[wid], dst_t, sg1)
  pltpu.async_copy(esc_hbm.at[wid], esc_t, sg1)

  def _z(i, carry):
    for j in range(HID // 16):
      rows0[i, pl.ds(j * 16, 16)] = _zeros16f()
    return carry
  lax.fori_loop(0, CH, _z, 0)

  @pl.when(sid < NS - 1)
  def _zero_full():
    for q in range(640 // CH):
      pltpu.async_copy(rows0, acc_sp.at[pl.ds(base + q * CH, CH)], sg0)
    for q in range(640 // CH):
      pltpu.make_async_copy(rows0, acc_sp.at[pl.ds(base + q * CH, CH)],
                            sg0).wait()

  @pl.when(sid == NS - 1)
  def _zero_last():
    for q in range(400 // CH):
      pltpu.async_copy(rows0, acc_sp.at[pl.ds(base + q * CH, CH)], sg0)
    for q in range(400 // CH):
      pltpu.make_async_copy(rows0, acc_sp.at[pl.ds(base + q * CH, CH)],
                            sg0).wait()

  pltpu.make_async_copy(gidx_hbm.at[wid], gidx_t, sg1).wait()
  pltpu.make_async_copy(dstp_hbm.at[wid], dst_t, sg1).wait()
  pltpu.make_async_copy(esc_hbm.at[wid], esc_t, sg1).wait()
  plsc.subcore_barrier()

  def _fire(c, rows, sem):
    pltpu.async_copy(table_hbm.at[gidx_t.at[c]], rows, sem)

  def _step(c, rows, sem):
    pltpu.make_async_copy(table_hbm.at[gidx_t.at[c]], rows, sem).wait()

    def _scale(g, carry):
      ev = esc_t[c, pl.ds(g * 16, 16)]
      for l in range(16):
        s = ev[l]
        r0 = g * 16 + l
        for j in range(HID // 16):
          rows[r0, pl.ds(j * 16, 16)] = rows[r0, pl.ds(j * 16, 16)] * s
      return carry
    lax.fori_loop(0, CH // 16, _scale, 0)

    pltpu.sync_copy(rows, acc_sp.at[dst_t.at[c]], add=True)

  _fire(0, rows0, sg0)
  _fire(1, rows1, sg1)

  def _pair(cc, carry):
    c0 = cc * 2
    _step(c0, rows0, sg0)
    _fire(c0 + 2, rows0, sg0)
    c1 = c0 + 1

    _step(c1, rows1, sg1)

    @pl.when(c1 + 2 < NCHUNK)
    def _fire_odd():
      _fire(c1 + 2, rows1, sg1)
    return carry
  lax.fori_loop(0, NCHUNK // 2, _pair, 0)
  _step(NCHUNK - 1, rows0, sg0)

  plsc.subcore_barrier()

  @pl.when(sid < NS - 1)
  def _drain_full():
    pltpu.sync_copy(acc_sp.at[pl.ds(base, 640)],
                    out_hbm.at[cid, pl.ds(base, 640)])

  @pl.when(sid == NS - 1)
  def _drain_last():
    pltpu.sync_copy(acc_sp.at[pl.ds(base, 400)],
                    out_hbm.at[cid, pl.ds(base, 400)])


NB = 10
BN = N // NB


def _tc1_body(a_ref, r1_ref, b1_ref, w2_ref, rt2_ref, xw_ref, xr_ref):
  x = jnp.maximum(a_ref[0] + a_ref[1] + r1_ref[...] + b1_ref[...], 0.0)
  xr_ref[...] = jnp.dot(x, rt2_ref[...], preferred_element_type=jnp.float32)
  for r in range(R):
    xw_ref[r] = jnp.dot(x, w2_ref[r], preferred_element_type=jnp.float32)


_tc1 = pl.pallas_call(
    _tc1_body,
    grid=(NB,),
    in_specs=[
        pl.BlockSpec((NC, BN, HID), lambda i: (0, i, 0)),
        pl.BlockSpec((BN, HID), lambda i: (i, 0)),
        pl.BlockSpec((1, HID), lambda i: (0, 0)),
        pl.BlockSpec((R, HID, OUT), lambda i: (0, 0, 0)),
        pl.BlockSpec((HID, OUT), lambda i: (0, 0)),
    ],
    out_specs=[
        pl.BlockSpec((R, BN, OUT), lambda i: (0, i, 0)),
        pl.BlockSpec((BN, OUT), lambda i: (i, 0)),
    ],
    out_shape=[
        jax.ShapeDtypeStruct((R, N, OUT), jnp.float32),
        jax.ShapeDtypeStruct((N, OUT), jnp.float32),
    ],
)


def _tc2_body(a_ref, xr_ref, b2_ref, o_ref):
  o_ref[...] = a_ref[0] + a_ref[1] + xr_ref[...] + b2_ref[...]


_tc2 = pl.pallas_call(
    _tc2_body,
    grid=(NB,),
    in_specs=[
        pl.BlockSpec((NC, BN, OUT), lambda i: (0, i, 0)),
        pl.BlockSpec((BN, OUT), lambda i: (i, 0)),
        pl.BlockSpec((1, OUT), lambda i: (0, 0)),
    ],
    out_specs=pl.BlockSpec((BN, OUT), lambda i: (i, 0)),
    out_shape=jax.ShapeDtypeStruct((N, OUT), jnp.float32),
)


def kernel(edge_index, edge_type, weight1, root1, bias1, weight2, root2, bias2):
  src = edge_index[0]
  dst = edge_index[1]
  gidx, dstp, esc = _prep(src, dst, edge_type)
  acc1 = _agg(weight1.reshape(R * N, HID), gidx, dstp, esc)
  xw, xr = _tc1(acc1, root1, bias1.reshape(1, HID), weight2, root2)
  acc2 = _agg(xw.reshape(R * N, OUT), gidx, dstp, esc)
  return _tc2(acc2, xr, bias2.reshape(1, OUT))
